# baseline (speedup 1.0000x reference)
import jax
import jax.numpy as jnp
from jax import lax
from jax.experimental import pallas as pl
from jax.experimental.pallas import tpu as pltpu
from jax.experimental.pallas import tpu_sc as plsc

NUM_BINS = 100000
NC, NS, L = 2, 16, 16
NW = NC * NS

ROWS, COLS = 16384, 26
CPW = ROWS // NW
HALF = CPW // 2
VSTEP = 2
STEPS = HALF // (L * VSTEP)

_C1 = 0x7FEB352D
_C2 = 0x846CA68B - (1 << 32)
_RECIP = float(1.0 / 3125.0)


def _srl(x, k):
    return lax.shift_right_logical(x, jnp.int32(k))


def _umin(a, b):
    return jnp.minimum(a, b)


def _hash_mod(x):
    x = x ^ _srl(x, 16)
    x = x * jnp.int32(_C1)
    x = x ^ _srl(x, 15)
    x = x * jnp.int32(_C2)
    h = x ^ _srl(x, 16)
    xs = _srl(h, 5)
    q = (xs.astype(jnp.float32) * jnp.float32(_RECIP)).astype(jnp.int32)
    t = h - q * jnp.int32(NUM_BINS) + jnp.int32(NUM_BINS)
    tu = lax.bitcast_convert_type(t, jnp.uint32)
    r = _umin(_umin(tu, tu - jnp.uint32(NUM_BINS)),
              tu - jnp.uint32(2 * NUM_BINS))
    return lax.bitcast_convert_type(r, jnp.int32)


def _compute(buf):
    def step(i, carry):
        c0 = i * (L * VSTEP)
        for r in range(COLS):
            for j in range(VSTEP):
                sl = (r, pl.ds(c0 + j * L, L))
                buf[sl] = _hash_mod(buf[sl])
        return carry

    lax.fori_loop(0, STEPS, step, 0)


def _sc_body(in_hbm, out_hbm, buf0, buf1, s0, s1, s2, s3):
    wid = lax.axis_index("s") * NC + lax.axis_index("c")
    base = wid * CPW
    cp0 = pltpu.async_copy(in_hbm.at[:, pl.ds(base, HALF)], buf0, s0)
    cp1 = pltpu.async_copy(in_hbm.at[:, pl.ds(base + HALF, HALF)], buf1, s1)
    cp0.wait()
    _compute(buf0)
    o0 = pltpu.async_copy(buf0, out_hbm.at[:, pl.ds(base, HALF)], s2)
    cp1.wait()
    _compute(buf1)
    o1 = pltpu.async_copy(buf1, out_hbm.at[:, pl.ds(base + HALF, HALF)], s3)
    o0.wait()
    o1.wait()


@jax.jit
def kernel(inputs):
    tin = inputs.T
    call = pl.kernel(
        _sc_body,
        out_type=jax.ShapeDtypeStruct((COLS, ROWS), jnp.int32),
        mesh=plsc.VectorSubcoreMesh(core_axis_name="c", subcore_axis_name="s"),
        scratch_types=[
            pltpu.VMEM((COLS, HALF), jnp.int32),
            pltpu.VMEM((COLS, HALF), jnp.int32),
            pltpu.SemaphoreType.DMA,
            pltpu.SemaphoreType.DMA,
            pltpu.SemaphoreType.DMA,
            pltpu.SemaphoreType.DMA,
        ],
    )
    return call(tin).T


# speedup vs baseline: 1.2618x; 1.2618x over previous
import jax
import jax.numpy as jnp
from jax import lax
from jax.experimental import pallas as pl
from jax.experimental.pallas import tpu as pltpu
from jax.experimental.pallas import tpu_sc as plsc

NUM_BINS = 100000
NC, NS, L = 2, 16, 16
NW = NC * NS

ROWS, COLS = 16384, 26
CPW = ROWS // NW
HALF = CPW // 2
VSTEP = 2
STEPS = HALF // (L * VSTEP)

_C1 = 0x7FEB352D
_C2 = 0x846CA68B - (1 << 32)
_RECIP = float(1.0 / 3125.0)


def _srl(x, k):
    return lax.shift_right_logical(x, jnp.int32(k))


def _umin(a, b):
    return jnp.minimum(a, b)


def _hash_mod(x):
    x = x ^ _srl(x, 16)
    x = x * jnp.int32(_C1)
    x = x ^ _srl(x, 15)
    x = x * jnp.int32(_C2)
    h = x ^ _srl(x, 16)
    xs = _srl(h, 5)
    q = (xs.astype(jnp.float32) * jnp.float32(_RECIP)).astype(jnp.int32)
    t = h - q * jnp.int32(NUM_BINS) + jnp.int32(NUM_BINS)
    tu = lax.bitcast_convert_type(t, jnp.uint32)
    r = _umin(_umin(tu, tu - jnp.uint32(NUM_BINS)),
              tu - jnp.uint32(2 * NUM_BINS))
    return lax.bitcast_convert_type(r, jnp.int32)


def _compute(buf):
    @plsc.parallel_loop(0, HALF, L, unroll=VSTEP)
    def body(c):
        for r in range(COLS):
            sl = (r, pl.ds(c, L))
            buf[sl] = _hash_mod(buf[sl])


def _sc_body(in_hbm, out_hbm, buf0, buf1, s0, s1, s2, s3):
    wid = lax.axis_index("s") * NC + lax.axis_index("c")
    base = wid * CPW
    cp0 = pltpu.async_copy(in_hbm.at[:, pl.ds(base, HALF)], buf0, s0)
    cp1 = pltpu.async_copy(in_hbm.at[:, pl.ds(base + HALF, HALF)], buf1, s1)
    cp0.wait()
    _compute(buf0)
    o0 = pltpu.async_copy(buf0, out_hbm.at[:, pl.ds(base, HALF)], s2)
    cp1.wait()
    _compute(buf1)
    o1 = pltpu.async_copy(buf1, out_hbm.at[:, pl.ds(base + HALF, HALF)], s3)
    o0.wait()
    o1.wait()


@jax.jit
def kernel(inputs):
    tin = inputs.T
    call = pl.kernel(
        _sc_body,
        out_type=jax.ShapeDtypeStruct((COLS, ROWS), jnp.int32),
        mesh=plsc.VectorSubcoreMesh(core_axis_name="c", subcore_axis_name="s"),
        scratch_types=[
            pltpu.VMEM((COLS, HALF), jnp.int32),
            pltpu.VMEM((COLS, HALF), jnp.int32),
            pltpu.SemaphoreType.DMA,
            pltpu.SemaphoreType.DMA,
            pltpu.SemaphoreType.DMA,
            pltpu.SemaphoreType.DMA,
        ],
    )
    return call(tin).T


# speedup vs baseline: 1.2642x; 1.0019x over previous
import jax
import jax.numpy as jnp
from jax import lax
from jax.experimental import pallas as pl
from jax.experimental.pallas import tpu as pltpu
from jax.experimental.pallas import tpu_sc as plsc

NUM_BINS = 100000
NC, NS, L = 2, 16, 16
NW = NC * NS

ROWS, COLS = 16384, 26
CPW = ROWS // NW
HALF = CPW // 2
VSTEP = 4
STEPS = HALF // (L * VSTEP)

_C1 = 0x7FEB352D
_C2 = 0x846CA68B - (1 << 32)
_RECIP = float(1.0 / 3125.0)


def _srl(x, k):
    return lax.shift_right_logical(x, jnp.int32(k))


def _umin(a, b):
    return jnp.minimum(a, b)


def _hash_mod(x):
    x = x ^ _srl(x, 16)
    x = x * jnp.int32(_C1)
    x = x ^ _srl(x, 15)
    x = x * jnp.int32(_C2)
    h = x ^ _srl(x, 16)
    xs = _srl(h, 5)
    q = (xs.astype(jnp.float32) * jnp.float32(_RECIP)).astype(jnp.int32)
    t = h - q * jnp.int32(NUM_BINS) + jnp.int32(NUM_BINS)
    tu = lax.bitcast_convert_type(t, jnp.uint32)
    r = _umin(_umin(tu, tu - jnp.uint32(NUM_BINS)),
              tu - jnp.uint32(2 * NUM_BINS))
    return lax.bitcast_convert_type(r, jnp.int32)


def _compute(buf):
    @plsc.parallel_loop(0, HALF, L, unroll=VSTEP)
    def body(c):
        for r in range(COLS):
            sl = (r, pl.ds(c, L))
            buf[sl] = _hash_mod(buf[sl])


def _sc_body(in_hbm, out_hbm, buf0, buf1, s0, s1, s2, s3):
    wid = lax.axis_index("s") * NC + lax.axis_index("c")
    base = wid * CPW
    cp0 = pltpu.async_copy(in_hbm.at[:, pl.ds(base, HALF)], buf0, s0)
    cp1 = pltpu.async_copy(in_hbm.at[:, pl.ds(base + HALF, HALF)], buf1, s1)
    cp0.wait()
    _compute(buf0)
    o0 = pltpu.async_copy(buf0, out_hbm.at[:, pl.ds(base, HALF)], s2)
    cp1.wait()
    _compute(buf1)
    o1 = pltpu.async_copy(buf1, out_hbm.at[:, pl.ds(base + HALF, HALF)], s3)
    o0.wait()
    o1.wait()


@jax.jit
def kernel(inputs):
    tin = inputs.T
    call = pl.kernel(
        _sc_body,
        out_type=jax.ShapeDtypeStruct((COLS, ROWS), jnp.int32),
        mesh=plsc.VectorSubcoreMesh(core_axis_name="c", subcore_axis_name="s"),
        scratch_types=[
            pltpu.VMEM((COLS, HALF), jnp.int32),
            pltpu.VMEM((COLS, HALF), jnp.int32),
            pltpu.SemaphoreType.DMA,
            pltpu.SemaphoreType.DMA,
            pltpu.SemaphoreType.DMA,
            pltpu.SemaphoreType.DMA,
        ],
    )
    return call(tin).T
